# baseline (speedup 1.0000x reference)
import jax
import jax.numpy as jnp
from jax.experimental import pallas as pl

N = 10000
E = 320000
HD = 128
H = 8
C = 16
G = 64


def _gat(x, edge_index, e, p):
    src = edge_index[0]
    dst = edge_index[1]
    n = x.shape[0]
    h = (x @ p['W']).reshape(n, H, C)
    ep = (e @ p['W_e']).reshape(-1, H, C)
    a_src = (h * p['att_src'][None]).sum(-1)
    a_dst = (h * p['att_dst'][None]).sum(-1)
    a_edge = (ep * p['att_edge'][None]).sum(-1)
    alpha = a_src[src] + a_dst[dst] + a_edge
    alpha = jax.nn.leaky_relu(alpha, 0.2)
    amax = jax.ops.segment_max(alpha, dst, num_segments=n)
    amax = jnp.where(jnp.isfinite(amax), amax, 0.0)
    ex = jnp.exp(alpha - amax[dst])
    den = jax.ops.segment_sum(ex, dst, num_segments=n)
    w = ex / (den[dst] + 1e-16)
    msg = h[src] * w[:, :, None]
    out = jax.ops.segment_sum(msg, dst, num_segments=n)
    return out.reshape(n, H * C) + p['bias']


def _mlp_pallas(g, W1, b1, W2, b2):
    def body(g_ref, w1_ref, b1_ref, w2_ref, b2_ref, o_ref):
        t = jnp.maximum(g_ref[...] @ w1_ref[...] + b1_ref[...][None], 0.0)
        o_ref[...] = jnp.maximum(t @ w2_ref[...] + b2_ref[...][None], 0.0)

    return pl.pallas_call(
        body,
        out_shape=jax.ShapeDtypeStruct((G, HD), jnp.float32),
    )(g, W1, b1, W2, b2)


def kernel(x, edge_index, edge_attr, batch, params):
    h = x @ params['node_W'] + params['node_b']
    e = edge_attr @ params['edge_W'] + params['edge_b']
    residual = h
    for i, p in enumerate(params['layers']):
        hn = _gat(h, edge_index, e, p)
        m = hn.mean(0)
        v = hn.var(0)
        hn = p['gamma'] * (hn - m) / jnp.sqrt(v + 1e-05) + p['beta']
        hn = jax.nn.relu(hn)
        if i > 0 and i % 2 == 0:
            hn = hn + residual
            residual = hn
        h = hn
    ones = jnp.ones((h.shape[0],), h.dtype)
    counts = jax.ops.segment_sum(ones, batch, num_segments=G)
    x_mean = jax.ops.segment_sum(h, batch, num_segments=G) / jnp.maximum(counts, 1.0)[:, None]
    x_max = jax.ops.segment_max(h, batch, num_segments=G)
    x_max = jnp.where(jnp.isfinite(x_max), x_max, 0.0)
    gate = jax.nn.relu(h @ params['gate_W1'] + params['gate_b1']) @ params['gate_W2'] + params['gate_b2']
    gmax = jax.ops.segment_max(gate, batch, num_segments=G)
    gmax = jnp.where(jnp.isfinite(gmax), gmax, 0.0)
    ge = jnp.exp(gate - gmax[batch])
    gden = jax.ops.segment_sum(ge, batch, num_segments=G)
    w = ge / (gden[batch] + 1e-16)
    x_att = jax.ops.segment_sum(w * h, batch, num_segments=G)
    g = jnp.concatenate([x_mean, x_max, x_att], axis=1)
    return _mlp_pallas(g, params['mlp_W1'], params['mlp_b1'],
                       params['mlp_W2'], params['mlp_b2'])


# speedup vs baseline: 36.2084x; 36.2084x over previous
import functools

import jax
import jax.numpy as jnp
from jax import lax
from jax.experimental import pallas as pl
from jax.experimental.pallas import tpu as pltpu
from jax.experimental.pallas import tpu_sc as plsc

N = 10000
E = 320000
HD = 128
H = 8
C = 16
L = 4
G = 64

NC = 2
NS = 16
NW = NC * NS
EPT = E // NW
CH = 80
NCHUNK = EPT // CH
RPT = N // NS
HT = HD + 16


def _edge_body(htab, adtab, aedge, src, dst, z128, z16,
               msg_out, den_out,
               msg_sh, den_sh,
               sidx_v, didx_v, ae_v, ad_v, hrows_v, msg_v, ex_v,
               sem1, sem2):
    cid = lax.axis_index("c")
    sid = lax.axis_index("s")
    wid = cid * NS + sid

    @pl.when(sid == 0)
    def _zero():
        pltpu.sync_copy(z128, msg_sh)
        pltpu.sync_copy(z16, den_sh)
    plsc.subcore_barrier()

    def chunk(j, _):
        base = pl.multiple_of(wid * EPT + j * CH, 8)
        cp_s = pltpu.async_copy(src.at[pl.ds(base, CH)], sidx_v, sem1)
        cp_d = pltpu.async_copy(dst.at[pl.ds(base, CH)], didx_v, sem2)
        cp_s.wait()
        cp_d.wait()
        cp_e = pltpu.async_copy(aedge.at[pl.ds(base, CH)], ae_v, sem1)
        cp_h = pltpu.async_copy(htab.at[sidx_v], hrows_v, sem2)
        cp_e.wait()
        cp_h.wait()
        cp_a = pltpu.async_copy(adtab.at[didx_v], ad_v, sem1)
        cp_a.wait()

        def edge(e, _):
            a = hrows_v[e, pl.ds(HD, 16)] + ad_v[e, :] + ae_v[e, :]
            a = jnp.where(a > 0.0, a, a * 0.2)
            ex = jnp.exp(a)
            ex_v[e, :] = ex
            for hh in range(H):
                sp = jnp.full((16,), ex[hh])
                msg_v[e, pl.ds(hh * 16, 16)] = (
                    hrows_v[e, pl.ds(hh * 16, 16)] * sp)
            return 0

        lax.fori_loop(0, CH, edge, 0)

        pltpu.sync_copy(msg_v, msg_sh.at[didx_v], add=True)
        pltpu.sync_copy(ex_v, den_sh.at[didx_v], add=True)
        return 0

    lax.fori_loop(0, NCHUNK, chunk, 0)
    plsc.subcore_barrier()

    @pl.when(sid == 0)
    def _writeback():
        pltpu.sync_copy(msg_sh, msg_out.at[cid])
        pltpu.sync_copy(den_sh, den_out.at[cid])


@functools.cache
def _edge_sc_kernel():
  return functools.partial(
    pl.kernel,
    out_type=(jax.ShapeDtypeStruct((NC, N, HD), jnp.float32),
              jax.ShapeDtypeStruct((NC, N, 16), jnp.float32)),
    mesh=plsc.VectorSubcoreMesh(core_axis_name="c", subcore_axis_name="s",
                                num_cores=NC, num_subcores=NS),
    compiler_params=pltpu.CompilerParams(use_tc_tiling_on_sc=False),
    scratch_types=[
        pltpu.VMEM_SHARED((N, HD), jnp.float32),
        pltpu.VMEM_SHARED((N, 16), jnp.float32),
        pltpu.VMEM((CH,), jnp.int32),
        pltpu.VMEM((CH,), jnp.int32),
        pltpu.VMEM((CH, 16), jnp.float32),
        pltpu.VMEM((CH, 16), jnp.float32),
        pltpu.VMEM((CH, HT), jnp.float32),
        pltpu.VMEM((CH, HD), jnp.float32),
        pltpu.VMEM((CH, 16), jnp.float32),
        pltpu.SemaphoreType.DMA,
        pltpu.SemaphoreType.DMA,
    ],
  )(_edge_body)


def _mm_body(x_ref, w_ref, o_ref):
    o_ref[...] = jnp.dot(x_ref[...], w_ref[...],
                         preferred_element_type=jnp.float32)


def _node_proj(xpad, wpad, b):
    def body(x_ref, w_ref, b_ref, o_ref):
        o_ref[...] = jnp.dot(x_ref[...], w_ref[...],
                             preferred_element_type=jnp.float32) + b_ref[...]
    return pl.pallas_call(
        body, out_shape=jax.ShapeDtypeStruct((N, HD), jnp.float32),
    )(xpad, wpad, b)


def _edge_proj(eap, k, k0):
    blk = 8000
    def body(e_ref, k_ref, k0_ref, o0, o1, o2, o3):
        full = jnp.dot(e_ref[...], k_ref[...],
                       preferred_element_type=jnp.float32) + k0_ref[...]
        o0[...] = full[:, 0:16]
        o1[...] = full[:, 16:32]
        o2[...] = full[:, 32:48]
        o3[...] = full[:, 48:64]
    outs = [jax.ShapeDtypeStruct((E, 16), jnp.float32)] * L
    return pl.pallas_call(
        body,
        grid=(E // blk,),
        in_specs=[pl.BlockSpec((blk, 8), lambda i: (i, 0)),
                  pl.BlockSpec((8, 64), lambda i: (0, 0)),
                  pl.BlockSpec((1, 64), lambda i: (0, 0))],
        out_specs=[pl.BlockSpec((blk, 16), lambda i: (i, 0))] * L,
        out_shape=outs,
    )(eap, k, k0)


def _layer_mm(h, wcat):
    blk = 2000
    def body(h_ref, w_ref, ht_ref, ad_ref):
        full = jnp.dot(h_ref[...], w_ref[...],
                       preferred_element_type=jnp.float32)
        ht_ref[...] = full[:, :HT]
        ad_ref[...] = full[:, HT:HT + 16]
    return pl.pallas_call(
        body,
        grid=(N // blk,),
        in_specs=[pl.BlockSpec((blk, HD), lambda i: (i, 0)),
                  pl.BlockSpec((HD, HT + 16), lambda i: (0, 0))],
        out_specs=[pl.BlockSpec((blk, HT), lambda i: (i, 0)),
                   pl.BlockSpec((blk, 16), lambda i: (i, 0))],
        out_shape=[jax.ShapeDtypeStruct((N, HT), jnp.float32),
                   jax.ShapeDtypeStruct((N, 16), jnp.float32)],
    )(h, wcat)


_CBLK = 2000


def _combine(msg2, den2, bias, gamma, beta, res, with_res):
    def body1(m_ref, d_ref, b_ref, o_ref, s_ref, q_ref):
        i = pl.program_id(0)
        num = m_ref[0] + m_ref[1]
        den = d_ref[0] + d_ref[1]
        d8 = den[:, :H]
        db = jnp.broadcast_to(d8[:, :, None], (_CBLK, H, C)).reshape(
            _CBLK, HD)
        out = jnp.where(db > 0.0, num / jnp.where(db > 0.0, db, 1.0), 0.0)
        out = out + b_ref[...]
        o_ref[...] = out

        @pl.when(i == 0)
        def _init():
            s_ref[...] = jnp.zeros_like(s_ref)
            q_ref[...] = jnp.zeros_like(q_ref)
        s_ref[...] += jnp.sum(out, axis=0, keepdims=True)
        q_ref[...] += jnp.sum(out * out, axis=0, keepdims=True)

    raw, s, q = pl.pallas_call(
        body1,
        grid=(N // _CBLK,),
        in_specs=[pl.BlockSpec((2, _CBLK, HD), lambda i: (0, i, 0)),
                  pl.BlockSpec((2, _CBLK, 16), lambda i: (0, i, 0)),
                  pl.BlockSpec((1, HD), lambda i: (0, 0))],
        out_specs=[pl.BlockSpec((_CBLK, HD), lambda i: (i, 0)),
                   pl.BlockSpec((1, HD), lambda i: (0, 0)),
                   pl.BlockSpec((1, HD), lambda i: (0, 0))],
        out_shape=[jax.ShapeDtypeStruct((N, HD), jnp.float32),
                   jax.ShapeDtypeStruct((1, HD), jnp.float32),
                   jax.ShapeDtypeStruct((1, HD), jnp.float32)],
    )(msg2, den2, bias)

    def body2(r_ref, s_ref, q_ref, g_ref, be_ref, h0_ref, o_ref):
        m = s_ref[...] / N
        v = q_ref[...] / N - m * m
        out = g_ref[...] * (r_ref[...] - m) * lax.rsqrt(v + 1e-05) \
            + be_ref[...]
        out = jnp.maximum(out, 0.0)
        if with_res:
            out = out + h0_ref[...]
        o_ref[...] = out

    return pl.pallas_call(
        body2,
        grid=(N // _CBLK,),
        in_specs=[pl.BlockSpec((_CBLK, HD), lambda i: (i, 0)),
                  pl.BlockSpec((1, HD), lambda i: (0, 0)),
                  pl.BlockSpec((1, HD), lambda i: (0, 0)),
                  pl.BlockSpec((1, HD), lambda i: (0, 0)),
                  pl.BlockSpec((1, HD), lambda i: (0, 0)),
                  pl.BlockSpec((_CBLK, HD), lambda i: (i, 0))],
        out_specs=pl.BlockSpec((_CBLK, HD), lambda i: (i, 0)),
        out_shape=jax.ShapeDtypeStruct((N, HD), jnp.float32),
    )(raw, s, q, gamma, beta, res)


def _pool_body(h_ref, b_ref, gw1, gb1, gw2, gb2, w1, b1, w2, b2, o_ref):
    h = h_ref[...]
    batch = b_ref[...]
    oh = (batch == lax.broadcasted_iota(jnp.int32, (1, G), 1)).astype(
        jnp.float32)
    ones = jnp.ones((N, 1), jnp.float32)
    dot_t = lambda a, b: lax.dot_general(
        a, b, (((0,), (0,)), ((), ())), preferred_element_type=jnp.float32)
    counts = dot_t(oh, ones)
    sums = dot_t(oh, h)
    x_mean = sums / jnp.maximum(counts, 1.0)

    g1 = jnp.maximum(jnp.dot(h, gw1[...],
                             preferred_element_type=jnp.float32)
                     + gb1[...], 0.0)
    gate = jnp.dot(g1, gw2[...], preferred_element_type=jnp.float32) \
        + gb2[...]

    xmax_rows = []
    gmax_rows = []
    for g in range(G):
        msk = batch == g
        xmax_rows.append(jnp.max(jnp.where(msk, h, -1e30), axis=0,
                                 keepdims=True))
        gmax_rows.append(jnp.max(jnp.where(msk, gate, -1e30), axis=0,
                                 keepdims=True))
    x_max = jnp.concatenate(xmax_rows, axis=0)
    x_max = jnp.where(counts > 0.0, x_max, 0.0)
    gmax = jnp.concatenate(gmax_rows, axis=0)
    gmax = jnp.where(counts > 0.0, gmax, 0.0)

    gmaxb = jnp.dot(oh, gmax, preferred_element_type=jnp.float32)
    ge = jnp.exp(gate - gmaxb)
    gden = dot_t(oh, ge)
    gdenb = jnp.dot(oh, gden, preferred_element_type=jnp.float32)
    w = ge / (gdenb + 1e-16)
    x_att = dot_t(oh, h * w[:, 0:1])

    gcat = jnp.concatenate([x_mean, x_max, x_att], axis=1)
    o = jnp.maximum(jnp.dot(gcat, w1[...],
                            preferred_element_type=jnp.float32) + b1[...],
                    0.0)
    o_ref[...] = jnp.maximum(
        jnp.dot(o, w2[...], preferred_element_type=jnp.float32) + b2[...],
        0.0)


def _pool(h, batch2, gw1, gb1, gw2p, gb2p, w1, b1, w2, b2):
    return pl.pallas_call(
        _pool_body, out_shape=jax.ShapeDtypeStruct((G, HD), jnp.float32),
    )(h, batch2, gw1, gb1, gw2p, gb2p, w1, b1, w2, b2)


def kernel(x, edge_index, edge_attr, batch, params):
    f32 = jnp.float32
    src = edge_index[0].astype(jnp.int32)
    dst = edge_index[1].astype(jnp.int32)
    batch2 = batch.astype(jnp.int32).reshape(N, 1)

    xpad = jnp.pad(x, ((0, 0), (0, 7)))
    wpad = jnp.pad(params['node_W'], ((0, 7), (0, 0)))
    nb = params['node_b'].reshape(1, HD)

    eap = jnp.pad(edge_attr, ((0, 0), (0, 5)))
    kmat = jnp.zeros((8, 4 * 16), f32)
    k0 = jnp.zeros((1, 4 * 16), f32)
    wcats = []
    for l, p in enumerate(params['layers']):
        w3 = p['W_e'].reshape(HD, H, C)
        me = (w3 * p['att_edge'][None]).sum(-1)
        kl = params['edge_W'] @ me
        k0l = params['edge_b'] @ me
        kmat = kmat.at[0:3, 16 * l:16 * l + 8].set(kl)
        k0 = k0.at[0, 16 * l:16 * l + 8].set(k0l)
        wl = p['W'].reshape(HD, H, C)
        msrc = (wl * p['att_src'][None]).sum(-1)
        mdst = (wl * p['att_dst'][None]).sum(-1)
        z8 = jnp.zeros((HD, 8), f32)
        wcats.append(jnp.concatenate([p['W'], msrc, z8, mdst, z8], axis=1))

    z128 = jnp.zeros((N, HD), f32)
    z16 = jnp.zeros((N, 16), f32)

    h = _node_proj(xpad, wpad, nb)
    aedges = _edge_proj(eap, kmat, k0)

    h0 = h
    for l, p in enumerate(params['layers']):
        htab, adtab = _layer_mm(h, wcats[l])
        msg2, den2 = _edge_sc_kernel()(htab, adtab, aedges[l], src, dst,
                                       z128, z16)
        h = _combine(msg2, den2, p['bias'].reshape(1, HD),
                     p['gamma'].reshape(1, HD), p['beta'].reshape(1, HD),
                     h0, l == 2)

    gw2p = jnp.pad(params['gate_W2'], ((0, 0), (0, 7)))
    gb2p = jnp.pad(params['gate_b2'], ((0, 7))).reshape(1, 8)
    return _pool(h, batch2, params['gate_W1'], params['gate_b1'].reshape(1, 64),
                 gw2p, gb2p, params['mlp_W1'], params['mlp_b1'].reshape(1, 256),
                 params['mlp_W2'], params['mlp_b2'].reshape(1, HD))
